# baseline (speedup 1.0000x reference)
import functools

import jax
import jax.numpy as jnp
from jax import lax
from jax.experimental import pallas as pl
from jax.experimental.pallas import tpu as pltpu
from jax.experimental.pallas import tpu_sc as plsc

LATENT = 64
LANES = 16
NUM_CORES = 2
NUM_SUBCORES = 16
NUM_WORKERS = NUM_CORES * NUM_SUBCORES
CHUNK = 128


@functools.partial(jax.jit, static_argnames=("batch",))
def _gmf_sc(uidx, iidx, user_table, item_table, wb, *, batch):
    bpw = batch // NUM_WORKERS
    nchunks = bpw // CHUNK

    mesh = plsc.VectorSubcoreMesh(core_axis_name="c", subcore_axis_name="s")

    @functools.partial(
        pl.kernel,
        out_type=jax.ShapeDtypeStruct((batch,), jnp.float32),
        mesh=mesh,
        compiler_params=pltpu.CompilerParams(
            needs_layout_passes=False, use_tc_tiling_on_sc=False),
        scratch_types=[
            pltpu.VMEM((nchunks, CHUNK), jnp.int32),
            pltpu.VMEM((nchunks, CHUNK), jnp.int32),
            pltpu.VMEM((nchunks, CHUNK, LATENT), jnp.float32),
            pltpu.VMEM((nchunks, CHUNK, LATENT), jnp.float32),
            pltpu.VMEM((LANES * 5,), jnp.float32),
            pltpu.VMEM((bpw,), jnp.float32),
            pltpu.SemaphoreType.DMA,
        ],
    )
    def gmf_kernel(uidx_hbm, iidx_hbm, utab_hbm, itab_hbm, wb_hbm, out_hbm,
                   uidx_v, iidx_v, urows_v, irows_v, wb_v, out_v, sem):
        wid = lax.axis_index("s") * NUM_CORES + lax.axis_index("c")
        base = wid * bpw

        pltpu.sync_copy(uidx_hbm.at[wid], uidx_v)
        pltpu.sync_copy(iidx_hbm.at[wid], iidx_v)
        pltpu.sync_copy(wb_hbm, wb_v)

        copies = []
        for j in range(nchunks):
            copies.append(pltpu.async_copy(
                utab_hbm.at[uidx_v.at[j]], urows_v.at[j], sem))
            copies.append(pltpu.async_copy(
                itab_hbm.at[iidx_v.at[j]], irows_v.at[j], sem))
        for cp in copies:
            cp.wait()

        w_chunks = [wb_v[pl.ds(c * LANES, LANES)] for c in range(LATENT // LANES)]
        bias = wb_v[pl.ds(LATENT, LANES)]
        last_lane = lax.iota(jnp.int32, LANES) == (LANES - 1)

        for j in range(nchunks):
            def row_body(r, _, j=j):
                acc = (urows_v[j, r, pl.ds(0, LANES)]
                       * irows_v[j, r, pl.ds(0, LANES)]) * w_chunks[0]
                for c in range(1, LATENT // LANES):
                    acc = acc + (urows_v[j, r, pl.ds(c * LANES, LANES)]
                                 * irows_v[j, r, pl.ds(c * LANES, LANES)]) * w_chunks[c]
                csum = plsc.cumsum(acc)
                pos = jnp.broadcast_to(j * CHUNK + r, (LANES,)).astype(jnp.int32)
                plsc.store_scatter(out_v, [pos], csum, mask=last_lane)
                return 0
            lax.fori_loop(0, CHUNK, row_body, 0)

        for i in range(bpw // LANES):
            x = out_v[pl.ds(i * LANES, LANES)]
            out_v[pl.ds(i * LANES, LANES)] = 1.0 / (1.0 + jnp.exp(-(x + bias)))

        pltpu.sync_copy(out_v, out_hbm.at[pl.ds(base, bpw)])

    return gmf_kernel(uidx, iidx, user_table, item_table, wb)


def kernel(user_indices, item_indices, user_table, item_table, W, b):
    batch = user_indices.shape[0]
    bpw = batch // NUM_WORKERS
    uidx = user_indices.astype(jnp.int32).reshape(NUM_WORKERS, bpw // CHUNK, CHUNK)
    iidx = item_indices.astype(jnp.int32).reshape(NUM_WORKERS, bpw // CHUNK, CHUNK)
    wb = jnp.concatenate([
        W.reshape(-1).astype(jnp.float32),
        jnp.broadcast_to(b.reshape(-1).astype(jnp.float32), (LANES,)),
    ])
    out = _gmf_sc(uidx, iidx, user_table, item_table, wb, batch=batch)
    return out.reshape(batch, 1)


# speedup vs baseline: 1.8260x; 1.8260x over previous
import functools

import jax
import jax.numpy as jnp
from jax import lax
from jax.experimental import pallas as pl
from jax.experimental.pallas import tpu as pltpu
from jax.experimental.pallas import tpu_sc as plsc

LATENT = 64
LANES = 16
NUM_CORES = 2
NUM_SUBCORES = 16
NUM_WORKERS = NUM_CORES * NUM_SUBCORES
WPT = NUM_SUBCORES
NBUF = 4
BSTART_W = 1056
DUMP = 1040

_params = pltpu.CompilerParams(
    needs_layout_passes=False, use_tc_tiling_on_sc=True,
    disable_bounds_checks=True)


def _build_meta(idx, rpw):
    B = idx.shape[0]
    perm = jnp.argsort(idx)
    s = idx[perm].astype(jnp.int32)
    tc = s >> 7
    r = jnp.arange(B, dtype=jnp.int32)
    newrun = jnp.concatenate(
        [jnp.ones((1,), jnp.bool_), tc[1:] != tc[:-1]])
    first = newrun | (r % rpw == 0)
    gblk = jnp.cumsum(first.astype(jnp.int32)) - 1
    gblk2 = gblk.reshape(WPT, rpw)
    local = gblk2 - gblk2[:, :1]
    nblk = local[:, -1] + 1
    wrow = r // rpw
    lcol = jnp.where(first, local.reshape(-1), DUMP)
    blk_tc = jnp.zeros((WPT, BSTART_W), jnp.int32).at[wrow, lcol].set(tc)
    blk_start = jnp.full((WPT, BSTART_W), rpw, jnp.int32).at[
        wrow, lcol].set(r % rpw)
    rank = jnp.zeros((B,), jnp.int32).at[perm].set(r)
    return dict(
        perm=perm, rank=rank, cols=(s & 127),
        blk_tc=blk_tc.reshape(WPT, BSTART_W // LANES, LANES),
        blk_start=blk_start.reshape(WPT, BSTART_W // LANES, LANES),
        nblk=jnp.broadcast_to(nblk[:, None], (WPT, LANES)),
        slot=local.reshape(WPT, rpw),
    )


@functools.partial(jax.jit, static_argnames=("batch",))
def _gmf_sc(utabT, itabT, umeta_cols, umeta_tc, umeta_start, umeta_nblk,
            vmeta_cols, vmeta_tc, vmeta_start, vmeta_nblk,
            vgidx, vpar, wb, *, batch):
    rpw = batch // WPT
    cpw = batch // NUM_WORKERS
    npair = batch // 2

    mesh = plsc.VectorSubcoreMesh(core_axis_name="c", subcore_axis_name="s")

    @functools.partial(
        pl.kernel,
        out_type=jax.ShapeDtypeStruct((2, npair, 2 * LATENT), jnp.float32),
        mesh=mesh,
        compiler_params=_params,
        scratch_types=[
            pltpu.VMEM((rpw // LANES, LANES), jnp.int32),
            pltpu.VMEM((BSTART_W // LANES, LANES), jnp.int32),
            pltpu.VMEM((BSTART_W // LANES, LANES), jnp.int32),
            pltpu.VMEM((LANES,), jnp.int32),
            pltpu.VMEM((NBUF, LATENT, 128), jnp.float32),
            pltpu.VMEM((rpw // 2, 2 * LATENT), jnp.float32),
            pltpu.SemaphoreType.DMA,
            pltpu.SemaphoreType.DMA,
            pltpu.SemaphoreType.DMA,
            pltpu.SemaphoreType.DMA,
            pltpu.SemaphoreType.DMA,
        ],
    )
    def phase_a(ucols_hbm, utc_hbm, ustart_hbm, unblk_hbm,
                vcols_hbm, vtc_hbm, vstart_hbm, vnblk_hbm,
                utabT_hbm, itabT_hbm, stage_hbm,
                cols_v, tc_v, start_v, nblk_v, ring_v, staged_v,
                s0, s1, s2, s3, sm):
        cid = lax.axis_index("c")
        sid = lax.axis_index("s")
        sems = [s0, s1, s2, s3]

        @pl.when(cid == 0)
        def _():
            pltpu.sync_copy(ucols_hbm.at[sid], cols_v)
            pltpu.sync_copy(utc_hbm.at[sid], tc_v)
            pltpu.sync_copy(ustart_hbm.at[sid], start_v)
            pltpu.sync_copy(unblk_hbm.at[sid], nblk_v)

        @pl.when(cid == 1)
        def _():
            pltpu.sync_copy(vcols_hbm.at[sid], cols_v)
            pltpu.sync_copy(vtc_hbm.at[sid], tc_v)
            pltpu.sync_copy(vstart_hbm.at[sid], start_v)
            pltpu.sync_copy(vnblk_hbm.at[sid], nblk_v)

        nblk = nblk_v[pl.ds(0, LANES)][0]
        lane_iota = lax.iota(jnp.int32, LANES)

        def sread(ref, i):
            vec = ref[i // LANES]
            return jnp.take(vec, jnp.broadcast_to(i % LANES, (LANES,)))[0]

        def fetch(b, k):
            tcb = sread(tc_v, b)
            colblk = pl.multiple_of(tcb * 128, 128)

            @pl.when(cid == 0)
            def _():
                pltpu.async_copy(
                    utabT_hbm.at[:, pl.ds(colblk, 128)], ring_v.at[k],
                    sems[k])

            @pl.when(cid == 1)
            def _():
                pltpu.async_copy(
                    itabT_hbm.at[:, pl.ds(colblk, 128)], ring_v.at[k],
                    sems[k])

        for k in range(NBUF):
            @pl.when(k < nblk)
            def _(k=k):
                fetch(k, k)

        d_iotas = [c * LANES + lane_iota for c in range(LATENT // LANES)]

        def round_body(g, _):
            for k in range(NBUF):
                b = g * NBUF + k

                @pl.when(b < nblk)
                def _(b=b, k=k):
                    pltpu.make_async_copy(
                        utabT_hbm.at[:, pl.ds(0, 128)], ring_v.at[k],
                        sems[k]).wait()
                    r0 = sread(start_v, b)
                    r1 = sread(start_v, b + 1)
                    kvec = jnp.broadcast_to(k, (LANES,))

                    def row_body(r, _, k=k, kvec=kvec):
                        c = sread(cols_v, r)
                        cvec = jnp.broadcast_to(c, (LANES,))
                        prow = jnp.broadcast_to(r // 2, (LANES,))
                        half = (r % 2) * LATENT
                        for ch in range(LATENT // LANES):
                            val = plsc.load_gather(
                                ring_v, [kvec, d_iotas[ch], cvec])
                            plsc.store_scatter(
                                staged_v,
                                [prow, half + ch * LANES + lane_iota], val)
                        return 0

                    lax.fori_loop(r0, r1, row_body, 0)
                    nxt = b + NBUF

                    @pl.when(nxt < nblk)
                    def _(nxt=nxt, k=k):
                        fetch(nxt, k)

            return 0

        lax.fori_loop(0, (nblk + NBUF - 1) // NBUF, round_body, 0)
        pltpu.sync_copy(
            staged_v,
            stage_hbm.at[cid, pl.ds(sid * (rpw // 2), rpw // 2), :])

    @functools.partial(
        pl.kernel,
        out_type=jax.ShapeDtypeStruct((batch,), jnp.float32),
        mesh=mesh,
        compiler_params=_params,
        scratch_types=[
            pltpu.VMEM((cpw // 2, 2 * LATENT), jnp.float32),
            pltpu.VMEM((NBUF, 128, 2 * LATENT), jnp.float32),
            pltpu.VMEM((NBUF, 128), jnp.int32),
            pltpu.VMEM((cpw // LANES, LANES), jnp.int32),
            pltpu.VMEM((LANES * 5,), jnp.float32),
            pltpu.VMEM((cpw,), jnp.float32),
            pltpu.SemaphoreType.DMA,
            pltpu.SemaphoreType.DMA,
        ],
    )
    def phase_c(stage_hbm, vgidx_hbm, vpar_hbm, wb_hbm, out_hbm,
                urows_v, vrows_v, gidx_v, par_v, wb_v, out_v, sg, sl):
        cid = lax.axis_index("c")
        sid = lax.axis_index("s")
        wid = sid * NUM_CORES + cid
        base = wid * cpw

        pltpu.sync_copy(vgidx_hbm.at[wid], gidx_v)
        pltpu.sync_copy(vpar_hbm.at[wid], par_v)
        pltpu.sync_copy(wb_hbm, wb_v)
        cp_u = pltpu.async_copy(
            stage_hbm.at[0, pl.ds(wid * (cpw // 2), cpw // 2), :],
            urows_v, sl)
        copies = [
            pltpu.async_copy(
                stage_hbm.at[1].at[gidx_v.at[j]], vrows_v.at[j], sg)
            for j in range(NBUF)
        ]
        cp_u.wait()
        for cp in copies:
            cp.wait()

        w_chunks = [wb_v[pl.ds(c * LANES, LANES)]
                    for c in range(LATENT // LANES)]
        bias = wb_v[pl.ds(LATENT, LANES)]
        lane_iota = lax.iota(jnp.int32, LANES)
        last_lane = lane_iota == (LANES - 1)

        def pair_body(p, _):
            parvec = par_v[p // (LANES // 2)]
            for u in range(2):
                r = p * 2 + u
                parb = jnp.take(
                    parvec, jnp.broadcast_to(r % LANES, (LANES,))) != 0
                vj = r // 128
                vr = r % 128
                acc = None
                for c in range(LATENT // LANES):
                    uvec = urows_v[p, pl.ds(u * LATENT + c * LANES, LANES)]
                    vlo = vrows_v[vj, vr, pl.ds(c * LANES, LANES)]
                    vhi = vrows_v[vj, vr, pl.ds(LATENT + c * LANES, LANES)]
                    vvec = jnp.where(parb, vhi, vlo)
                    term = (uvec * vvec) * w_chunks[c]
                    acc = term if acc is None else acc + term
                csum = plsc.cumsum(acc)
                pos = jnp.broadcast_to(r, (LANES,)).astype(jnp.int32)
                plsc.store_scatter(out_v, [pos], csum, mask=last_lane)
            return 0

        lax.fori_loop(0, cpw // 2, pair_body, 0)

        for i in range(cpw // LANES):
            x = out_v[pl.ds(i * LANES, LANES)]
            out_v[pl.ds(i * LANES, LANES)] = 1.0 / (1.0 + jnp.exp(-(x + bias)))

        pltpu.sync_copy(out_v, out_hbm.at[pl.ds(base, cpw)])

    stage = phase_a(umeta_cols, umeta_tc, umeta_start, umeta_nblk,
                    vmeta_cols, vmeta_tc, vmeta_start, vmeta_nblk,
                    utabT, itabT)
    return phase_c(stage, vgidx, vpar, wb)


def kernel(user_indices, item_indices, user_table, item_table, W, b):
    batch = user_indices.shape[0]
    rpw = batch // WPT
    cpw = batch // NUM_WORKERS
    um = _build_meta(user_indices.astype(jnp.int32), rpw)
    vm = _build_meta(item_indices.astype(jnp.int32), rpw)
    vr = vm["rank"][um["perm"]]
    vgidx = (vr >> 1).reshape(NUM_WORKERS, NBUF, 128)
    vpar = (vr & 1).reshape(NUM_WORKERS, cpw // LANES, LANES)
    ucols = um["cols"].reshape(WPT, rpw // LANES, LANES)
    vcols = vm["cols"].reshape(WPT, rpw // LANES, LANES)
    wb = jnp.concatenate([
        W.reshape(-1).astype(jnp.float32),
        jnp.broadcast_to(b.reshape(-1).astype(jnp.float32), (LANES,)),
    ])
    logits_sorted = _gmf_sc(
        user_table.T, item_table.T,
        ucols, um["blk_tc"], um["blk_start"], um["nblk"],
        vcols, vm["blk_tc"], vm["blk_start"], vm["nblk"],
        vgidx, vpar, wb, batch=batch)
    out = logits_sorted[um["rank"]]
    return out.reshape(batch, 1)


# speedup vs baseline: 3.4843x; 1.9081x over previous
import functools

import jax
import jax.numpy as jnp
from jax import lax
from jax.experimental import pallas as pl
from jax.experimental.pallas import tpu as pltpu
from jax.experimental.pallas import tpu_sc as plsc

LATENT = 64
LANES = 16
NUM_CORES = 2
NUM_SUBCORES = 16
NUM_WORKERS = NUM_CORES * NUM_SUBCORES
WPT = NUM_SUBCORES
NBUF = 4
BSTART_W = 1056
DUMP = 1040

_params = pltpu.CompilerParams(
    needs_layout_passes=False, use_tc_tiling_on_sc=True,
    disable_bounds_checks=True)


def _build_meta(idx, rpw):
    B = idx.shape[0]
    perm = jnp.argsort(idx)
    s = idx[perm].astype(jnp.int32)
    tc = s >> 7
    r = jnp.arange(B, dtype=jnp.int32)
    newrun = jnp.concatenate(
        [jnp.ones((1,), jnp.bool_), tc[1:] != tc[:-1]])
    first = newrun | (r % rpw == 0)
    gblk = jnp.cumsum(first.astype(jnp.int32)) - 1
    gblk2 = gblk.reshape(WPT, rpw)
    local = gblk2 - gblk2[:, :1]
    nblk = local[:, -1] + 1
    bb = jnp.arange(BSTART_W, dtype=jnp.int32)
    blk_start = (local[:, None, :] < bb[None, :, None]).astype(
        jnp.int32).sum(-1)
    tc2 = tc.reshape(WPT, rpw)
    blk_tc = jnp.take_along_axis(
        tc2, jnp.minimum(blk_start, rpw - 1), axis=1)
    rank = jnp.argsort(perm)
    return dict(
        perm=perm, rank=rank, cols=(s & 127),
        blk_tc=blk_tc.reshape(WPT, BSTART_W // LANES, LANES),
        blk_start=blk_start.reshape(WPT, BSTART_W // LANES, LANES),
        nblk=jnp.broadcast_to(nblk[:, None], (WPT, LANES)),
    )


@functools.partial(jax.jit, static_argnames=("batch",))
def _gmf_sc(utabT, itabT, umeta_cols, umeta_tc, umeta_start, umeta_nblk,
            vmeta_cols, vmeta_tc, vmeta_start, vmeta_nblk,
            vgidx, vpar, wb, *, batch):
    rpw = batch // WPT
    cpw = batch // NUM_WORKERS
    npair = batch // 2

    mesh = plsc.VectorSubcoreMesh(core_axis_name="c", subcore_axis_name="s")

    @functools.partial(
        pl.kernel,
        out_type=jax.ShapeDtypeStruct((2, npair, 2 * LATENT), jnp.float32),
        mesh=mesh,
        compiler_params=_params,
        scratch_types=[
            pltpu.VMEM((rpw // LANES, LANES), jnp.int32),
            pltpu.VMEM((BSTART_W // LANES, LANES), jnp.int32),
            pltpu.VMEM((BSTART_W // LANES, LANES), jnp.int32),
            pltpu.VMEM((LANES,), jnp.int32),
            pltpu.VMEM((NBUF, LATENT, 128), jnp.float32),
            pltpu.VMEM((rpw // 2, 2 * LATENT), jnp.float32),
            pltpu.SemaphoreType.DMA,
            pltpu.SemaphoreType.DMA,
            pltpu.SemaphoreType.DMA,
            pltpu.SemaphoreType.DMA,
            pltpu.SemaphoreType.DMA,
        ],
    )
    def phase_a(ucols_hbm, utc_hbm, ustart_hbm, unblk_hbm,
                vcols_hbm, vtc_hbm, vstart_hbm, vnblk_hbm,
                utabT_hbm, itabT_hbm, stage_hbm,
                cols_v, tc_v, start_v, nblk_v, ring_v, staged_v,
                s0, s1, s2, s3, sm):
        cid = lax.axis_index("c")
        sid = lax.axis_index("s")
        sems = [s0, s1, s2, s3]

        @pl.when(cid == 0)
        def _():
            pltpu.sync_copy(ucols_hbm.at[sid], cols_v)
            pltpu.sync_copy(utc_hbm.at[sid], tc_v)
            pltpu.sync_copy(ustart_hbm.at[sid], start_v)
            pltpu.sync_copy(unblk_hbm.at[sid], nblk_v)

        @pl.when(cid == 1)
        def _():
            pltpu.sync_copy(vcols_hbm.at[sid], cols_v)
            pltpu.sync_copy(vtc_hbm.at[sid], tc_v)
            pltpu.sync_copy(vstart_hbm.at[sid], start_v)
            pltpu.sync_copy(vnblk_hbm.at[sid], nblk_v)

        nblk = nblk_v[pl.ds(0, LANES)][0]
        lane_iota = lax.iota(jnp.int32, LANES)

        def sread(ref, i):
            vec = ref[i // LANES]
            return jnp.take(vec, jnp.broadcast_to(i % LANES, (LANES,)))[0]

        def fetch(b, k):
            tcb = sread(tc_v, b)
            colblk = pl.multiple_of(tcb * 128, 128)

            @pl.when(cid == 0)
            def _():
                pltpu.async_copy(
                    utabT_hbm.at[:, pl.ds(colblk, 128)], ring_v.at[k],
                    sems[k])

            @pl.when(cid == 1)
            def _():
                pltpu.async_copy(
                    itabT_hbm.at[:, pl.ds(colblk, 128)], ring_v.at[k],
                    sems[k])

        for k in range(NBUF):
            @pl.when(k < nblk)
            def _(k=k):
                fetch(k, k)

        d_iotas = [c * LANES + lane_iota for c in range(LATENT // LANES)]

        def round_body(g, _):
            for k in range(NBUF):
                b = g * NBUF + k

                @pl.when(b < nblk)
                def _(b=b, k=k):
                    pltpu.make_async_copy(
                        utabT_hbm.at[:, pl.ds(0, 128)], ring_v.at[k],
                        sems[k]).wait()
                    r0 = sread(start_v, b)
                    r1 = sread(start_v, b + 1)
                    kvec = jnp.broadcast_to(k, (LANES,))

                    def row_body(r, _, k=k, kvec=kvec):
                        c = sread(cols_v, r)
                        cvec = jnp.broadcast_to(c, (LANES,))
                        prow = jnp.broadcast_to(r // 2, (LANES,))
                        half = (r % 2) * LATENT
                        for ch in range(LATENT // LANES):
                            val = plsc.load_gather(
                                ring_v, [kvec, d_iotas[ch], cvec])
                            plsc.store_scatter(
                                staged_v,
                                [prow, half + ch * LANES + lane_iota], val)
                        return 0

                    lax.fori_loop(r0, r1, row_body, 0)
                    nxt = b + NBUF

                    @pl.when(nxt < nblk)
                    def _(nxt=nxt, k=k):
                        fetch(nxt, k)

            return 0

        lax.fori_loop(0, (nblk + NBUF - 1) // NBUF, round_body, 0)
        pltpu.sync_copy(
            staged_v,
            stage_hbm.at[cid, pl.ds(sid * (rpw // 2), rpw // 2), :])

    @functools.partial(
        pl.kernel,
        out_type=jax.ShapeDtypeStruct((batch,), jnp.float32),
        mesh=mesh,
        compiler_params=_params,
        scratch_types=[
            pltpu.VMEM((cpw // 2, 2 * LATENT), jnp.float32),
            pltpu.VMEM((NBUF, 128, 2 * LATENT), jnp.float32),
            pltpu.VMEM((NBUF, 128), jnp.int32),
            pltpu.VMEM((cpw // LANES, LANES), jnp.int32),
            pltpu.VMEM((LANES * 5,), jnp.float32),
            pltpu.VMEM((cpw,), jnp.float32),
            pltpu.SemaphoreType.DMA,
            pltpu.SemaphoreType.DMA,
        ],
    )
    def phase_c(stage_hbm, vgidx_hbm, vpar_hbm, wb_hbm, out_hbm,
                urows_v, vrows_v, gidx_v, par_v, wb_v, out_v, sg, sl):
        cid = lax.axis_index("c")
        sid = lax.axis_index("s")
        wid = sid * NUM_CORES + cid
        base = wid * cpw

        pltpu.sync_copy(vgidx_hbm.at[wid], gidx_v)
        pltpu.sync_copy(vpar_hbm.at[wid], par_v)
        pltpu.sync_copy(wb_hbm, wb_v)
        cp_u = pltpu.async_copy(
            stage_hbm.at[0, pl.ds(wid * (cpw // 2), cpw // 2), :],
            urows_v, sl)
        copies = [
            pltpu.async_copy(
                stage_hbm.at[1].at[gidx_v.at[j]], vrows_v.at[j], sg)
            for j in range(NBUF)
        ]
        cp_u.wait()
        for cp in copies:
            cp.wait()

        w_chunks = [wb_v[pl.ds(c * LANES, LANES)]
                    for c in range(LATENT // LANES)]
        bias = wb_v[pl.ds(LATENT, LANES)]
        lane_iota = lax.iota(jnp.int32, LANES)
        last_lane = lane_iota == (LANES - 1)

        def pair_body(p, _):
            parvec = par_v[p // (LANES // 2)]
            for u in range(2):
                r = p * 2 + u
                parb = jnp.take(
                    parvec, jnp.broadcast_to(r % LANES, (LANES,))) != 0
                vj = r // 128
                vr = r % 128
                acc = None
                for c in range(LATENT // LANES):
                    uvec = urows_v[p, pl.ds(u * LATENT + c * LANES, LANES)]
                    vlo = vrows_v[vj, vr, pl.ds(c * LANES, LANES)]
                    vhi = vrows_v[vj, vr, pl.ds(LATENT + c * LANES, LANES)]
                    vvec = jnp.where(parb, vhi, vlo)
                    term = (uvec * vvec) * w_chunks[c]
                    acc = term if acc is None else acc + term
                csum = plsc.cumsum(acc)
                pos = jnp.broadcast_to(r, (LANES,)).astype(jnp.int32)
                plsc.store_scatter(out_v, [pos], csum, mask=last_lane)
            return 0

        lax.fori_loop(0, cpw // 2, pair_body, 0)

        for i in range(cpw // LANES):
            x = out_v[pl.ds(i * LANES, LANES)]
            out_v[pl.ds(i * LANES, LANES)] = 1.0 / (1.0 + jnp.exp(-(x + bias)))

        pltpu.sync_copy(out_v, out_hbm.at[pl.ds(base, cpw)])

    stage = phase_a(umeta_cols, umeta_tc, umeta_start, umeta_nblk,
                    vmeta_cols, vmeta_tc, vmeta_start, vmeta_nblk,
                    utabT, itabT)
    return phase_c(stage, vgidx, vpar, wb)


def kernel(user_indices, item_indices, user_table, item_table, W, b):
    batch = user_indices.shape[0]
    rpw = batch // WPT
    cpw = batch // NUM_WORKERS
    um = _build_meta(user_indices.astype(jnp.int32), rpw)
    vm = _build_meta(item_indices.astype(jnp.int32), rpw)
    vr = vm["rank"][um["perm"]]
    vgidx = (vr >> 1).reshape(NUM_WORKERS, NBUF, 128)
    vpar = (vr & 1).reshape(NUM_WORKERS, cpw // LANES, LANES)
    ucols = um["cols"].reshape(WPT, rpw // LANES, LANES)
    vcols = vm["cols"].reshape(WPT, rpw // LANES, LANES)
    wb = jnp.concatenate([
        W.reshape(-1).astype(jnp.float32),
        jnp.broadcast_to(b.reshape(-1).astype(jnp.float32), (LANES,)),
    ])
    logits_sorted = _gmf_sc(
        user_table.T, item_table.T,
        ucols, um["blk_tc"], um["blk_start"], um["nblk"],
        vcols, vm["blk_tc"], vm["blk_start"], vm["nblk"],
        vgidx, vpar, wb, batch=batch)
    out = logits_sorted[um["rank"]]
    return out.reshape(batch, 1)


# speedup vs baseline: 3.6696x; 1.0532x over previous
import functools

import jax
import jax.numpy as jnp
from jax import lax
from jax.experimental import pallas as pl
from jax.experimental.pallas import tpu as pltpu
from jax.experimental.pallas import tpu_sc as plsc

LATENT = 64
LANES = 16
NUM_CORES = 2
NUM_SUBCORES = 16
NUM_WORKERS = NUM_CORES * NUM_SUBCORES
WPT = NUM_SUBCORES
NBUF = 4
ABUF = 4
BSTART_W = 1056
DUMP = 1040

_params = pltpu.CompilerParams(
    needs_layout_passes=False, use_tc_tiling_on_sc=True,
    disable_bounds_checks=True)


def _build_meta(idx, rpw):
    B = idx.shape[0]
    r = jnp.arange(B, dtype=jnp.int32)
    s, perm = lax.sort_key_val(idx.astype(jnp.int32), r)
    tc = s >> 7
    newrun = jnp.concatenate(
        [jnp.ones((1,), jnp.bool_), tc[1:] != tc[:-1]])
    first = newrun | (r % rpw == 0)
    gblk = jnp.cumsum(first.astype(jnp.int32)) - 1
    gblk2 = gblk.reshape(WPT, rpw)
    local = gblk2 - gblk2[:, :1]
    nblk = local[:, -1] + 1
    bb = jnp.arange(BSTART_W, dtype=jnp.int32)
    blk_start = (local[:, None, :] < bb[None, :, None]).astype(
        jnp.int32).sum(-1)
    tc2 = tc.reshape(WPT, rpw)
    blk_tc = jnp.take_along_axis(
        tc2, jnp.minimum(blk_start, rpw - 1), axis=1)
    rank = lax.sort_key_val(perm, r)[1]
    return dict(
        perm=perm, rank=rank, cols=(s & 127),
        blk_tc=blk_tc.reshape(WPT, BSTART_W // LANES, LANES),
        blk_start=blk_start.reshape(WPT, BSTART_W // LANES, LANES),
        nblk=jnp.broadcast_to(nblk[:, None], (WPT, LANES)),
    )


@functools.partial(jax.jit, static_argnames=("batch",))
def _gmf_sc(utabT, itabT, umeta_cols, umeta_tc, umeta_start, umeta_nblk,
            vmeta_cols, vmeta_tc, vmeta_start, vmeta_nblk,
            vgidx, vpar, wb, *, batch):
    rpw = batch // WPT
    cpw = batch // NUM_WORKERS
    npair = batch // 2

    mesh = plsc.VectorSubcoreMesh(core_axis_name="c", subcore_axis_name="s")

    @functools.partial(
        pl.kernel,
        out_type=jax.ShapeDtypeStruct((2, npair, 2 * LATENT), jnp.float32),
        mesh=mesh,
        compiler_params=_params,
        scratch_types=[
            pltpu.VMEM((rpw // LANES, LANES), jnp.int32),
            pltpu.VMEM((BSTART_W // LANES, LANES), jnp.int32),
            pltpu.VMEM((BSTART_W // LANES, LANES), jnp.int32),
            pltpu.VMEM((LANES,), jnp.int32),
            pltpu.VMEM((ABUF, LATENT, 128), jnp.float32),
            pltpu.VMEM((rpw // 2, 2 * LATENT), jnp.float32),
        ] + [pltpu.SemaphoreType.DMA] * (ABUF + 1),
    )
    def phase_a(ucols_hbm, utc_hbm, ustart_hbm, unblk_hbm,
                vcols_hbm, vtc_hbm, vstart_hbm, vnblk_hbm,
                utabT_hbm, itabT_hbm, stage_hbm,
                cols_v, tc_v, start_v, nblk_v, ring_v, staged_v,
                *sems):
        cid = lax.axis_index("c")
        sid = lax.axis_index("s")

        @pl.when(cid == 0)
        def _():
            pltpu.sync_copy(ucols_hbm.at[sid], cols_v)
            pltpu.sync_copy(utc_hbm.at[sid], tc_v)
            pltpu.sync_copy(ustart_hbm.at[sid], start_v)
            pltpu.sync_copy(unblk_hbm.at[sid], nblk_v)

        @pl.when(cid == 1)
        def _():
            pltpu.sync_copy(vcols_hbm.at[sid], cols_v)
            pltpu.sync_copy(vtc_hbm.at[sid], tc_v)
            pltpu.sync_copy(vstart_hbm.at[sid], start_v)
            pltpu.sync_copy(vnblk_hbm.at[sid], nblk_v)

        nblk = nblk_v[pl.ds(0, LANES)][0]
        lane_iota = lax.iota(jnp.int32, LANES)

        def sread(ref, i):
            vec = ref[i // LANES]
            return jnp.take(vec, jnp.broadcast_to(i % LANES, (LANES,)))[0]

        def fetch(b, k):
            tcb = sread(tc_v, b)
            colblk = pl.multiple_of(tcb * 128, 128)

            @pl.when(cid == 0)
            def _():
                pltpu.async_copy(
                    utabT_hbm.at[:, pl.ds(colblk, 128)], ring_v.at[k],
                    sems[k])

            @pl.when(cid == 1)
            def _():
                pltpu.async_copy(
                    itabT_hbm.at[:, pl.ds(colblk, 128)], ring_v.at[k],
                    sems[k])

        for k in range(ABUF):
            @pl.when(k < nblk)
            def _(k=k):
                fetch(k, k)

        d_iotas = [c * LANES + lane_iota for c in range(LATENT // LANES)]

        def round_body(g, _):
            for k in range(ABUF):
                b = g * ABUF + k

                @pl.when(b < nblk)
                def _(b=b, k=k):
                    pltpu.make_async_copy(
                        utabT_hbm.at[:, pl.ds(0, 128)], ring_v.at[k],
                        sems[k]).wait()
                    r0 = sread(start_v, b)
                    r1 = sread(start_v, b + 1)
                    kvec = jnp.broadcast_to(k, (LANES,))

                    def row_body(r, _, k=k, kvec=kvec):
                        c = sread(cols_v, r)
                        cvec = jnp.broadcast_to(c, (LANES,))
                        prow = jnp.broadcast_to(r // 2, (LANES,))
                        half = (r % 2) * LATENT
                        for ch in range(LATENT // LANES):
                            val = plsc.load_gather(
                                ring_v, [kvec, d_iotas[ch], cvec])
                            plsc.store_scatter(
                                staged_v,
                                [prow, half + ch * LANES + lane_iota], val)
                        return 0

                    lax.fori_loop(r0, r1, row_body, 0)
                    nxt = b + ABUF

                    @pl.when(nxt < nblk)
                    def _(nxt=nxt, k=k):
                        fetch(nxt, k)

            return 0

        lax.fori_loop(0, (nblk + ABUF - 1) // ABUF, round_body, 0)
        pltpu.sync_copy(
            staged_v,
            stage_hbm.at[cid, pl.ds(sid * (rpw // 2), rpw // 2), :])

    @functools.partial(
        pl.kernel,
        out_type=jax.ShapeDtypeStruct((batch,), jnp.float32),
        mesh=mesh,
        compiler_params=_params,
        scratch_types=[
            pltpu.VMEM((cpw // 2, 2 * LATENT), jnp.float32),
            pltpu.VMEM((NBUF, 128, 2 * LATENT), jnp.float32),
            pltpu.VMEM((NBUF, 128), jnp.int32),
            pltpu.VMEM((cpw // LANES, LANES), jnp.int32),
            pltpu.VMEM((LANES * 5,), jnp.float32),
            pltpu.VMEM((cpw,), jnp.float32),
            pltpu.SemaphoreType.DMA,
            pltpu.SemaphoreType.DMA,
        ],
    )
    def phase_c(stage_hbm, vgidx_hbm, vpar_hbm, wb_hbm, out_hbm,
                urows_v, vrows_v, gidx_v, par_v, wb_v, out_v, sg, sl):
        cid = lax.axis_index("c")
        sid = lax.axis_index("s")
        wid = sid * NUM_CORES + cid
        base = wid * cpw

        pltpu.sync_copy(vgidx_hbm.at[wid], gidx_v)
        pltpu.sync_copy(vpar_hbm.at[wid], par_v)
        pltpu.sync_copy(wb_hbm, wb_v)
        cp_u = pltpu.async_copy(
            stage_hbm.at[0, pl.ds(wid * (cpw // 2), cpw // 2), :],
            urows_v, sl)
        copies = [
            pltpu.async_copy(
                stage_hbm.at[1].at[gidx_v.at[j]], vrows_v.at[j], sg)
            for j in range(NBUF)
        ]
        cp_u.wait()
        for cp in copies:
            cp.wait()

        w_chunks = [wb_v[pl.ds(c * LANES, LANES)]
                    for c in range(LATENT // LANES)]
        bias = wb_v[pl.ds(LATENT, LANES)]
        lane_iota = lax.iota(jnp.int32, LANES)
        last_lane = lane_iota == (LANES - 1)

        def pair_body(p, _):
            parvec = par_v[p // (LANES // 2)]
            for u in range(2):
                r = p * 2 + u
                parb = jnp.take(
                    parvec, jnp.broadcast_to(r % LANES, (LANES,))) != 0
                vj = r // 128
                vr = r % 128
                acc = None
                for c in range(LATENT // LANES):
                    uvec = urows_v[p, pl.ds(u * LATENT + c * LANES, LANES)]
                    vlo = vrows_v[vj, vr, pl.ds(c * LANES, LANES)]
                    vhi = vrows_v[vj, vr, pl.ds(LATENT + c * LANES, LANES)]
                    vvec = jnp.where(parb, vhi, vlo)
                    term = (uvec * vvec) * w_chunks[c]
                    acc = term if acc is None else acc + term
                csum = plsc.cumsum(acc)
                pos = jnp.broadcast_to(r, (LANES,)).astype(jnp.int32)
                plsc.store_scatter(out_v, [pos], csum, mask=last_lane)
            return 0

        lax.fori_loop(0, cpw // 2, pair_body, 0)

        for i in range(cpw // LANES):
            x = out_v[pl.ds(i * LANES, LANES)]
            out_v[pl.ds(i * LANES, LANES)] = 1.0 / (1.0 + jnp.exp(-(x + bias)))

        pltpu.sync_copy(out_v, out_hbm.at[pl.ds(base, cpw)])

    stage = phase_a(umeta_cols, umeta_tc, umeta_start, umeta_nblk,
                    vmeta_cols, vmeta_tc, vmeta_start, vmeta_nblk,
                    utabT, itabT)
    return phase_c(stage, vgidx, vpar, wb)


def kernel(user_indices, item_indices, user_table, item_table, W, b):
    batch = user_indices.shape[0]
    rpw = batch // WPT
    cpw = batch // NUM_WORKERS
    um = _build_meta(user_indices.astype(jnp.int32), rpw)
    vm = _build_meta(item_indices.astype(jnp.int32), rpw)
    vr = vm["rank"][um["perm"]]
    vgidx = (vr >> 1).reshape(NUM_WORKERS, NBUF, 128)
    vpar = (vr & 1).reshape(NUM_WORKERS, cpw // LANES, LANES)
    ucols = um["cols"].reshape(WPT, rpw // LANES, LANES)
    vcols = vm["cols"].reshape(WPT, rpw // LANES, LANES)
    wb = jnp.concatenate([
        W.reshape(-1).astype(jnp.float32),
        jnp.broadcast_to(b.reshape(-1).astype(jnp.float32), (LANES,)),
    ])
    logits_sorted = _gmf_sc(
        user_table.T, item_table.T,
        ucols, um["blk_tc"], um["blk_start"], um["nblk"],
        vcols, vm["blk_tc"], vm["blk_start"], vm["nblk"],
        vgidx, vpar, wb, batch=batch)
    out = logits_sorted[um["rank"]]
    return out.reshape(batch, 1)


# speedup vs baseline: 3.9253x; 1.0697x over previous
import functools

import jax
import jax.numpy as jnp
from jax import lax
from jax.experimental import pallas as pl
from jax.experimental.pallas import tpu as pltpu
from jax.experimental.pallas import tpu_sc as plsc

LATENT = 64
LANES = 16
NUM_CORES = 2
NUM_SUBCORES = 16
NUM_WORKERS = NUM_CORES * NUM_SUBCORES
WPT = NUM_WORKERS
NBUF = 4
ABUF = 6
BSTART_W = 544

_params = pltpu.CompilerParams(
    needs_layout_passes=False, use_tc_tiling_on_sc=True,
    disable_bounds_checks=True)


def _build_meta(idx, rpw):
    B = idx.shape[0]
    r = jnp.arange(B, dtype=jnp.int32)
    s, perm = lax.sort_key_val(idx.astype(jnp.int32), r)
    tc = s >> 7
    newrun = jnp.concatenate(
        [jnp.ones((1,), jnp.bool_), tc[1:] != tc[:-1]])
    first = newrun | (r % rpw == 0)
    gblk = jnp.cumsum(first.astype(jnp.int32)) - 1
    gblk2 = gblk.reshape(WPT, rpw)
    local = gblk2 - gblk2[:, :1]
    nblk = local[:, -1] + 1
    bb = jnp.arange(BSTART_W, dtype=jnp.int32)
    blk_start = (local[:, None, :] < bb[None, :, None]).astype(
        jnp.int32).sum(-1)
    tc2 = tc.reshape(WPT, rpw)
    blk_tc = jnp.take_along_axis(
        tc2, jnp.minimum(blk_start, rpw - 1), axis=1)
    rank = lax.sort_key_val(perm, r)[1]
    return dict(
        perm=perm, rank=rank, cols=(s & 127),
        blk_tc=blk_tc.reshape(WPT, BSTART_W // LANES, LANES),
        blk_start=blk_start.reshape(WPT, BSTART_W // LANES, LANES),
        nblk=jnp.broadcast_to(nblk[:, None], (WPT, LANES)),
    )


@functools.partial(jax.jit, static_argnames=("batch",))
def _gmf_sc(utabT, itabT, umeta_cols, umeta_tc, umeta_start, umeta_nblk,
            vmeta_cols, vmeta_tc, vmeta_start, vmeta_nblk,
            vgidx, vpar, wb, *, batch):
    rpw = batch // WPT
    cpw = batch // NUM_WORKERS
    npair = batch // 2

    mesh = plsc.VectorSubcoreMesh(core_axis_name="c", subcore_axis_name="s")

    @functools.partial(
        pl.kernel,
        out_type=jax.ShapeDtypeStruct((npair, 2 * LATENT), jnp.float32),
        mesh=mesh,
        compiler_params=_params,
        scratch_types=[
            pltpu.VMEM((rpw // LANES, LANES), jnp.int32),
            pltpu.VMEM((BSTART_W // LANES, LANES), jnp.int32),
            pltpu.VMEM((BSTART_W // LANES, LANES), jnp.int32),
            pltpu.VMEM((LANES,), jnp.int32),
            pltpu.VMEM((ABUF, LATENT, 128), jnp.float32),
            pltpu.VMEM((rpw // 2, 2 * LATENT), jnp.float32),
        ] + [pltpu.SemaphoreType.DMA] * (ABUF + 1),
    )
    def phase_a(cols_hbm, tc_hbm, start_hbm, nblk_hbm, tabT_hbm, stage_hbm,
                cols_v, tc_v, start_v, nblk_v, ring_v, staged_v, *sems):
        cid = lax.axis_index("c")
        sid = lax.axis_index("s")
        wid = sid * NUM_CORES + cid

        pltpu.sync_copy(cols_hbm.at[wid], cols_v)
        pltpu.sync_copy(tc_hbm.at[wid], tc_v)
        pltpu.sync_copy(start_hbm.at[wid], start_v)
        pltpu.sync_copy(nblk_hbm.at[wid], nblk_v)

        nblk = nblk_v[pl.ds(0, LANES)][0]
        lane_iota = lax.iota(jnp.int32, LANES)

        def sread(ref, i):
            vec = ref[i // LANES]
            return jnp.take(vec, jnp.broadcast_to(i % LANES, (LANES,)))[0]

        def fetch(b, k):
            tcb = sread(tc_v, b)
            colblk = pl.multiple_of(tcb * 128, 128)
            pltpu.async_copy(
                tabT_hbm.at[:, pl.ds(colblk, 128)], ring_v.at[k], sems[k])

        for k in range(ABUF):
            @pl.when(k < nblk)
            def _(k=k):
                fetch(k, k)

        d_iotas = [c * LANES + lane_iota for c in range(LATENT // LANES)]

        def round_body(g, _):
            for k in range(ABUF):
                b = g * ABUF + k

                @pl.when(b < nblk)
                def _(b=b, k=k):
                    pltpu.make_async_copy(
                        tabT_hbm.at[:, pl.ds(0, 128)], ring_v.at[k],
                        sems[k]).wait()
                    r0 = sread(start_v, b)
                    r1 = sread(start_v, b + 1)
                    kvec = jnp.broadcast_to(k, (LANES,))

                    def row_body(r, _, k=k, kvec=kvec):
                        c = sread(cols_v, r)
                        cvec = jnp.broadcast_to(c, (LANES,))
                        prow = jnp.broadcast_to(r // 2, (LANES,))
                        half = (r % 2) * LATENT
                        for ch in range(LATENT // LANES):
                            val = plsc.load_gather(
                                ring_v, [kvec, d_iotas[ch], cvec])
                            plsc.store_scatter(
                                staged_v,
                                [prow, half + ch * LANES + lane_iota], val)
                        return 0

                    lax.fori_loop(r0, r1, row_body, 0)
                    nxt = b + ABUF

                    @pl.when(nxt < nblk)
                    def _(nxt=nxt, k=k):
                        fetch(nxt, k)

            return 0

        lax.fori_loop(0, (nblk + ABUF - 1) // ABUF, round_body, 0)
        pltpu.sync_copy(
            staged_v,
            stage_hbm.at[pl.ds(wid * (rpw // 2), rpw // 2), :])

    @functools.partial(
        pl.kernel,
        out_type=jax.ShapeDtypeStruct((batch,), jnp.float32),
        mesh=mesh,
        compiler_params=_params,
        scratch_types=[
            pltpu.VMEM((cpw // 2, 2 * LATENT), jnp.float32),
            pltpu.VMEM((NBUF, 128, 2 * LATENT), jnp.float32),
            pltpu.VMEM((NBUF, 128), jnp.int32),
            pltpu.VMEM((cpw // LANES, LANES), jnp.int32),
            pltpu.VMEM((LANES * 5,), jnp.float32),
            pltpu.VMEM((cpw,), jnp.float32),
            pltpu.SemaphoreType.DMA,
            pltpu.SemaphoreType.DMA,
        ],
    )
    def phase_c(stage_u_hbm, stage_v_hbm, vgidx_hbm, vpar_hbm, wb_hbm,
                out_hbm,
                urows_v, vrows_v, gidx_v, par_v, wb_v, out_v, sg, sl):
        cid = lax.axis_index("c")
        sid = lax.axis_index("s")
        wid = sid * NUM_CORES + cid
        base = wid * cpw

        pltpu.sync_copy(vgidx_hbm.at[wid], gidx_v)
        pltpu.sync_copy(vpar_hbm.at[wid], par_v)
        pltpu.sync_copy(wb_hbm, wb_v)
        cp_u = pltpu.async_copy(
            stage_u_hbm.at[pl.ds(wid * (cpw // 2), cpw // 2), :],
            urows_v, sl)
        copies = [
            pltpu.async_copy(
                stage_v_hbm.at[gidx_v.at[j]], vrows_v.at[j], sg)
            for j in range(NBUF)
        ]
        cp_u.wait()
        for cp in copies:
            cp.wait()

        w_chunks = [wb_v[pl.ds(c * LANES, LANES)]
                    for c in range(LATENT // LANES)]
        bias = wb_v[pl.ds(LATENT, LANES)]
        lane_iota = lax.iota(jnp.int32, LANES)
        last_lane = lane_iota == (LANES - 1)

        def pair_body(p, _):
            parvec = par_v[p // (LANES // 2)]
            for u in range(2):
                r = p * 2 + u
                parb = jnp.take(
                    parvec, jnp.broadcast_to(r % LANES, (LANES,))) != 0
                vj = r // 128
                vr = r % 128
                acc = None
                for c in range(LATENT // LANES):
                    uvec = urows_v[p, pl.ds(u * LATENT + c * LANES, LANES)]
                    vlo = vrows_v[vj, vr, pl.ds(c * LANES, LANES)]
                    vhi = vrows_v[vj, vr, pl.ds(LATENT + c * LANES, LANES)]
                    vvec = jnp.where(parb, vhi, vlo)
                    term = (uvec * vvec) * w_chunks[c]
                    acc = term if acc is None else acc + term
                csum = plsc.cumsum(acc)
                pos = jnp.broadcast_to(r, (LANES,)).astype(jnp.int32)
                plsc.store_scatter(out_v, [pos], csum, mask=last_lane)
            return 0

        lax.fori_loop(0, cpw // 2, pair_body, 0)

        for i in range(cpw // LANES):
            x = out_v[pl.ds(i * LANES, LANES)]
            out_v[pl.ds(i * LANES, LANES)] = 1.0 / (1.0 + jnp.exp(-(x + bias)))

        pltpu.sync_copy(out_v, out_hbm.at[pl.ds(base, cpw)])

    stage_u = phase_a(umeta_cols, umeta_tc, umeta_start, umeta_nblk, utabT)
    stage_v = phase_a(vmeta_cols, vmeta_tc, vmeta_start, vmeta_nblk, itabT)
    return phase_c(stage_u, stage_v, vgidx, vpar, wb)


def kernel(user_indices, item_indices, user_table, item_table, W, b):
    batch = user_indices.shape[0]
    rpw = batch // WPT
    cpw = batch // NUM_WORKERS
    um = _build_meta(user_indices.astype(jnp.int32), rpw)
    vm = _build_meta(item_indices.astype(jnp.int32), rpw)
    vr = vm["rank"][um["perm"]]
    vgidx = (vr >> 1).reshape(NUM_WORKERS, NBUF, 128)
    vpar = (vr & 1).reshape(NUM_WORKERS, cpw // LANES, LANES)
    ucols = um["cols"].reshape(WPT, rpw // LANES, LANES)
    vcols = vm["cols"].reshape(WPT, rpw // LANES, LANES)
    wb = jnp.concatenate([
        W.reshape(-1).astype(jnp.float32),
        jnp.broadcast_to(b.reshape(-1).astype(jnp.float32), (LANES,)),
    ])
    logits_sorted = _gmf_sc(
        user_table.T, item_table.T,
        ucols, um["blk_tc"], um["blk_start"], um["nblk"],
        vcols, vm["blk_tc"], vm["blk_start"], vm["nblk"],
        vgidx, vpar, wb, batch=batch)
    out = logits_sorted[um["rank"]]
    return out.reshape(batch, 1)


# speedup vs baseline: 3.9461x; 1.0053x over previous
import functools

import jax
import jax.numpy as jnp
from jax import lax
from jax.experimental import pallas as pl
from jax.experimental.pallas import tpu as pltpu
from jax.experimental.pallas import tpu_sc as plsc

LATENT = 64
LANES = 16
NUM_CORES = 2
NUM_SUBCORES = 16
NUM_WORKERS = NUM_CORES * NUM_SUBCORES
WPT = NUM_WORKERS
NBUF = 4
ABUF = 8
BSTART_W = 544

_params = pltpu.CompilerParams(
    needs_layout_passes=False, use_tc_tiling_on_sc=True,
    disable_bounds_checks=True)


def _build_meta(idx, rpw):
    B = idx.shape[0]
    r = jnp.arange(B, dtype=jnp.int32)
    s, perm = lax.sort_key_val(idx.astype(jnp.int32), r)
    tc = s >> 7
    newrun = jnp.concatenate(
        [jnp.ones((1,), jnp.bool_), tc[1:] != tc[:-1]])
    first = newrun | (r % rpw == 0)
    gblk = jnp.cumsum(first.astype(jnp.int32)) - 1
    gblk2 = gblk.reshape(WPT, rpw)
    local = gblk2 - gblk2[:, :1]
    nblk = local[:, -1] + 1
    bb = jnp.arange(BSTART_W, dtype=jnp.int32)
    blk_start = (local[:, None, :] < bb[None, :, None]).astype(
        jnp.int32).sum(-1)
    tc2 = tc.reshape(WPT, rpw)
    blk_tc = jnp.take_along_axis(
        tc2, jnp.minimum(blk_start, rpw - 1), axis=1)
    rank = lax.sort_key_val(perm, r)[1]
    return dict(
        perm=perm, rank=rank, cols=(s & 127),
        blk_tc=blk_tc.reshape(WPT, BSTART_W // LANES, LANES),
        blk_start=blk_start.reshape(WPT, BSTART_W // LANES, LANES),
        nblk=jnp.broadcast_to(nblk[:, None], (WPT, LANES)),
    )


@functools.partial(jax.jit, static_argnames=("batch",))
def _gmf_sc(utabT, itabT, umeta_cols, umeta_tc, umeta_start, umeta_nblk,
            vmeta_cols, vmeta_tc, vmeta_start, vmeta_nblk,
            vgidx, vpar, wb, *, batch):
    rpw = batch // WPT
    cpw = batch // NUM_WORKERS
    npair = batch // 2

    mesh = plsc.VectorSubcoreMesh(core_axis_name="c", subcore_axis_name="s")

    @functools.partial(
        pl.kernel,
        out_type=jax.ShapeDtypeStruct((npair, 2 * LATENT), jnp.float32),
        mesh=mesh,
        compiler_params=_params,
        scratch_types=[
            pltpu.VMEM((rpw // LANES, LANES), jnp.int32),
            pltpu.VMEM((BSTART_W // LANES, LANES), jnp.int32),
            pltpu.VMEM((BSTART_W // LANES, LANES), jnp.int32),
            pltpu.VMEM((LANES,), jnp.int32),
            pltpu.VMEM((ABUF, LATENT, 128), jnp.float32),
            pltpu.VMEM((rpw // 2, 2 * LATENT), jnp.float32),
        ] + [pltpu.SemaphoreType.DMA] * (ABUF + 1),
    )
    def phase_a(cols_hbm, tc_hbm, start_hbm, nblk_hbm, tabT_hbm, stage_hbm,
                cols_v, tc_v, start_v, nblk_v, ring_v, staged_v, *sems):
        cid = lax.axis_index("c")
        sid = lax.axis_index("s")
        wid = sid * NUM_CORES + cid

        pltpu.sync_copy(cols_hbm.at[wid], cols_v)
        pltpu.sync_copy(tc_hbm.at[wid], tc_v)
        pltpu.sync_copy(start_hbm.at[wid], start_v)
        pltpu.sync_copy(nblk_hbm.at[wid], nblk_v)

        nblk = nblk_v[pl.ds(0, LANES)][0]
        lane_iota = lax.iota(jnp.int32, LANES)

        def sread(ref, i):
            vec = ref[i // LANES]
            return jnp.take(vec, jnp.broadcast_to(i % LANES, (LANES,)))[0]

        def fetch(b, k):
            tcb = sread(tc_v, b)
            colblk = pl.multiple_of(tcb * 128, 128)
            pltpu.async_copy(
                tabT_hbm.at[:, pl.ds(colblk, 128)], ring_v.at[k], sems[k])

        for k in range(ABUF):
            @pl.when(k < nblk)
            def _(k=k):
                fetch(k, k)

        d_iotas = [c * LANES + lane_iota for c in range(LATENT // LANES)]

        def round_body(g, _):
            for k in range(ABUF):
                b = g * ABUF + k

                @pl.when(b < nblk)
                def _(b=b, k=k):
                    pltpu.make_async_copy(
                        tabT_hbm.at[:, pl.ds(0, 128)], ring_v.at[k],
                        sems[k]).wait()
                    r0 = sread(start_v, b)
                    r1 = sread(start_v, b + 1)
                    kvec = jnp.broadcast_to(k, (LANES,))

                    def row_body(r, _, k=k, kvec=kvec):
                        c = sread(cols_v, r)
                        cvec = jnp.broadcast_to(c, (LANES,))
                        prow = jnp.broadcast_to(r // 2, (LANES,))
                        half = (r % 2) * LATENT
                        for ch in range(LATENT // LANES):
                            val = plsc.load_gather(
                                ring_v, [kvec, d_iotas[ch], cvec])
                            plsc.store_scatter(
                                staged_v,
                                [prow, half + ch * LANES + lane_iota], val)
                        return 0

                    lax.fori_loop(r0, r1, row_body, 0)
                    nxt = b + ABUF

                    @pl.when(nxt < nblk)
                    def _(nxt=nxt, k=k):
                        fetch(nxt, k)

            return 0

        lax.fori_loop(0, (nblk + ABUF - 1) // ABUF, round_body, 0)
        pltpu.sync_copy(
            staged_v,
            stage_hbm.at[pl.ds(wid * (rpw // 2), rpw // 2), :])

    @functools.partial(
        pl.kernel,
        out_type=jax.ShapeDtypeStruct((batch,), jnp.float32),
        mesh=mesh,
        compiler_params=_params,
        scratch_types=[
            pltpu.VMEM((cpw // 2, 2 * LATENT), jnp.float32),
            pltpu.VMEM((NBUF, 128, 2 * LATENT), jnp.float32),
            pltpu.VMEM((NBUF, 128), jnp.int32),
            pltpu.VMEM((cpw // LANES, LANES), jnp.int32),
            pltpu.VMEM((LANES * 5,), jnp.float32),
            pltpu.VMEM((cpw,), jnp.float32),
            pltpu.SemaphoreType.DMA,
            pltpu.SemaphoreType.DMA,
        ],
    )
    def phase_c(stage_u_hbm, stage_v_hbm, vgidx_hbm, vpar_hbm, wb_hbm,
                out_hbm,
                urows_v, vrows_v, gidx_v, par_v, wb_v, out_v, sg, sl):
        cid = lax.axis_index("c")
        sid = lax.axis_index("s")
        wid = sid * NUM_CORES + cid
        base = wid * cpw

        pltpu.sync_copy(vgidx_hbm.at[wid], gidx_v)
        pltpu.sync_copy(vpar_hbm.at[wid], par_v)
        pltpu.sync_copy(wb_hbm, wb_v)
        cp_u = pltpu.async_copy(
            stage_u_hbm.at[pl.ds(wid * (cpw // 2), cpw // 2), :],
            urows_v, sl)
        copies = [
            pltpu.async_copy(
                stage_v_hbm.at[gidx_v.at[j]], vrows_v.at[j], sg)
            for j in range(NBUF)
        ]
        cp_u.wait()
        for cp in copies:
            cp.wait()

        w_chunks = [wb_v[pl.ds(c * LANES, LANES)]
                    for c in range(LATENT // LANES)]
        bias = wb_v[pl.ds(LATENT, LANES)]
        lane_iota = lax.iota(jnp.int32, LANES)
        last_lane = lane_iota == (LANES - 1)

        def pair_body(p, _):
            parvec = par_v[p // (LANES // 2)]
            for u in range(2):
                r = p * 2 + u
                parb = jnp.take(
                    parvec, jnp.broadcast_to(r % LANES, (LANES,))) != 0
                vj = r // 128
                vr = r % 128
                acc = None
                for c in range(LATENT // LANES):
                    uvec = urows_v[p, pl.ds(u * LATENT + c * LANES, LANES)]
                    vlo = vrows_v[vj, vr, pl.ds(c * LANES, LANES)]
                    vhi = vrows_v[vj, vr, pl.ds(LATENT + c * LANES, LANES)]
                    vvec = jnp.where(parb, vhi, vlo)
                    term = (uvec * vvec) * w_chunks[c]
                    acc = term if acc is None else acc + term
                csum = plsc.cumsum(acc)
                pos = jnp.broadcast_to(r, (LANES,)).astype(jnp.int32)
                plsc.store_scatter(out_v, [pos], csum, mask=last_lane)
            return 0

        lax.fori_loop(0, cpw // 2, pair_body, 0)

        for i in range(cpw // LANES):
            x = out_v[pl.ds(i * LANES, LANES)]
            out_v[pl.ds(i * LANES, LANES)] = 1.0 / (1.0 + jnp.exp(-(x + bias)))

        pltpu.sync_copy(out_v, out_hbm.at[pl.ds(base, cpw)])

    stage_u = phase_a(umeta_cols, umeta_tc, umeta_start, umeta_nblk, utabT)
    stage_v = phase_a(vmeta_cols, vmeta_tc, vmeta_start, vmeta_nblk, itabT)
    return phase_c(stage_u, stage_v, vgidx, vpar, wb)


def kernel(user_indices, item_indices, user_table, item_table, W, b):
    batch = user_indices.shape[0]
    rpw = batch // WPT
    cpw = batch // NUM_WORKERS
    um = _build_meta(user_indices.astype(jnp.int32), rpw)
    vm = _build_meta(item_indices.astype(jnp.int32), rpw)
    vr = vm["rank"][um["perm"]]
    vgidx = (vr >> 1).reshape(NUM_WORKERS, NBUF, 128)
    vpar = (vr & 1).reshape(NUM_WORKERS, cpw // LANES, LANES)
    ucols = um["cols"].reshape(WPT, rpw // LANES, LANES)
    vcols = vm["cols"].reshape(WPT, rpw // LANES, LANES)
    wb = jnp.concatenate([
        W.reshape(-1).astype(jnp.float32),
        jnp.broadcast_to(b.reshape(-1).astype(jnp.float32), (LANES,)),
    ])
    logits_sorted = _gmf_sc(
        user_table.T, item_table.T,
        ucols, um["blk_tc"], um["blk_start"], um["nblk"],
        vcols, vm["blk_tc"], vm["blk_start"], vm["nblk"],
        vgidx, vpar, wb, batch=batch)
    out = logits_sorted[um["rank"]]
    return out.reshape(batch, 1)
